# Initial kernel scaffold; baseline (speedup 1.0000x reference)
import jax
import jax.numpy as jnp
from jax.experimental import pallas as pl


def kernel(x, modality_index, emb_table):
    raise NotImplementedError("write your pallas kernel here")


# speedup vs baseline: 4.5740x; 4.5740x over previous
import jax
import jax.numpy as jnp
from jax.experimental import pallas as pl
from jax.experimental.pallas import tpu as pltpu

B, S, D = 4, 4096, 1024
NUM_EMB = 4

ROWS = B * S
BLK = 2048


def _bcast_kernel(idx_ref, table_ref, out_ref):
    idx = idx_ref[0]
    row_ids = jax.lax.broadcasted_iota(jnp.int32, (NUM_EMB, D), 0)
    row = jnp.sum(jnp.where(row_ids == idx, table_ref[...], 0.0),
                  axis=0, keepdims=True)
    out_ref[...] = jnp.broadcast_to(row, out_ref.shape)


def kernel(x, modality_index, emb_table):
    del x
    idx = jnp.asarray(modality_index, jnp.int32).reshape((1,))
    out = pl.pallas_call(
        _bcast_kernel,
        grid_spec=pltpu.PrefetchScalarGridSpec(
            num_scalar_prefetch=1,
            grid=(ROWS // BLK,),
            in_specs=[pl.BlockSpec((NUM_EMB, D), lambda i, *_: (0, 0))],
            out_specs=pl.BlockSpec((BLK, D), lambda i, *_: (i, 0)),
        ),
        out_shape=jax.ShapeDtypeStruct((ROWS, D), jnp.float32),
    )(idx, emb_table)
    return out.reshape(B, S, D)
